# Initial kernel scaffold; baseline (speedup 1.0000x reference)
import jax
import jax.numpy as jnp
from jax.experimental import pallas as pl


def kernel(word_embeddings, token_type_ids, type_embeddings, position_embeddings, gamma, beta):
    raise NotImplementedError("write your pallas kernel here")


# speedup vs baseline: 2.6674x; 2.6674x over previous
import jax
import jax.numpy as jnp
from jax.experimental import pallas as pl

_EPS = 1e-12


def _fused_body(ids_ref, word_ref, pos_ref, type_ref, gamma_ref, beta_ref, out_ref):
    ids = ids_ref[0, 0, :]
    t = ids.shape[0]
    v = type_ref.shape[0]
    onehot = (ids[:, None] == jax.lax.broadcasted_iota(jnp.int32, (t, v), 1)
              ).astype(jnp.float32)
    typ = jnp.dot(onehot, type_ref[...], preferred_element_type=jnp.float32)
    x = word_ref[0] + pos_ref[...] + typ
    mean = jnp.mean(x, axis=-1, keepdims=True)
    cent = x - mean
    var = jnp.mean(cent * cent, axis=-1, keepdims=True)
    normed = cent * jax.lax.rsqrt(var + _EPS)
    out_ref[0] = normed * gamma_ref[0][None, :] + beta_ref[0][None, :]


def kernel(word_embeddings, token_type_ids, type_embeddings, position_embeddings,
           gamma, beta):
    b, s, d = word_embeddings.shape
    v = type_embeddings.shape[0]
    blk = 512
    nblk = s // blk

    ids3 = token_type_ids.astype(jnp.int32).reshape(b * nblk, 1, blk)
    pos = position_embeddings[:s]
    gamma2 = gamma.reshape(1, d)
    beta2 = beta.reshape(1, d)

    out = pl.pallas_call(
        _fused_body,
        grid=(b, nblk),
        in_specs=[
            pl.BlockSpec((1, 1, blk), lambda i, j, n=nblk: (i * n + j, 0, 0)),
            pl.BlockSpec((1, blk, d), lambda i, j: (i, j, 0)),
            pl.BlockSpec((blk, d), lambda i, j: (j, 0)),
            pl.BlockSpec((v, d), lambda i, j: (0, 0)),
            pl.BlockSpec((1, d), lambda i, j: (0, 0)),
            pl.BlockSpec((1, d), lambda i, j: (0, 0)),
        ],
        out_specs=pl.BlockSpec((1, blk, d), lambda i, j: (i, j, 0)),
        out_shape=jax.ShapeDtypeStruct((b, s, d), jnp.float32),
    )(ids3, word_embeddings, pos, type_embeddings, gamma2, beta2)
    return out


# speedup vs baseline: 2.9552x; 1.1079x over previous
import jax
import jax.numpy as jnp
from jax.experimental import pallas as pl

_EPS = 1e-12


def _fused_body(ids_ref, word_ref, pos_ref, type_ref, gamma_ref, beta_ref, out_ref):
    ids = ids_ref[0, 0, :]
    t = ids.shape[0]
    v = type_ref.shape[0]
    onehot = (ids[:, None] == jax.lax.broadcasted_iota(jnp.int32, (t, v), 1)
              ).astype(jnp.float32)
    typ = jnp.dot(onehot, type_ref[...], preferred_element_type=jnp.float32)
    x = word_ref[0] + pos_ref[...] + typ
    mean = jnp.mean(x, axis=-1, keepdims=True)
    cent = x - mean
    var = jnp.mean(cent * cent, axis=-1, keepdims=True)
    normed = cent * jax.lax.rsqrt(var + _EPS)
    out_ref[0] = normed * gamma_ref[0][None, :] + beta_ref[0][None, :]


def kernel(word_embeddings, token_type_ids, type_embeddings, position_embeddings,
           gamma, beta):
    b, s, d = word_embeddings.shape
    v = type_embeddings.shape[0]
    blk = 512
    nblk = s // blk

    ids3 = token_type_ids.astype(jnp.int32).reshape(b * nblk, 1, blk)
    pos = position_embeddings[:s]
    gamma2 = gamma.reshape(1, d)
    beta2 = beta.reshape(1, d)

    out = pl.pallas_call(
        _fused_body,
        grid=(nblk, b),
        in_specs=[
            pl.BlockSpec((1, 1, blk), lambda j, i, n=nblk: (i * n + j, 0, 0)),
            pl.BlockSpec((1, blk, d), lambda j, i: (i, j, 0)),
            pl.BlockSpec((blk, d), lambda j, i: (j, 0)),
            pl.BlockSpec((v, d), lambda j, i: (0, 0)),
            pl.BlockSpec((1, d), lambda j, i: (0, 0)),
            pl.BlockSpec((1, d), lambda j, i: (0, 0)),
        ],
        out_specs=pl.BlockSpec((1, blk, d), lambda j, i: (i, j, 0)),
        out_shape=jax.ShapeDtypeStruct((b, s, d), jnp.float32),
    )(ids3, word_embeddings, pos, type_embeddings, gamma2, beta2)
    return out


# speedup vs baseline: 3.2436x; 1.0976x over previous
import jax
import jax.numpy as jnp
from jax.experimental import pallas as pl

_EPS = 1e-12


def _fused_body(ids_ref, word_ref, pos_ref, type_ref, gamma_ref, beta_ref, out_ref):
    ids = ids_ref[0, 0, :]
    t = ids.shape[0]
    v = type_ref.shape[0]
    onehot = (ids[:, None] == jax.lax.broadcasted_iota(jnp.int32, (t, v), 1)
              ).astype(jnp.float32)
    typ = jnp.dot(onehot, type_ref[...], preferred_element_type=jnp.float32)
    x = word_ref[0] + pos_ref[...] + typ
    mean = jnp.mean(x, axis=-1, keepdims=True)
    cent = x - mean
    var = jnp.mean(cent * cent, axis=-1, keepdims=True)
    normed = cent * jax.lax.rsqrt(var + _EPS)
    out_ref[0] = normed * gamma_ref[0][None, :] + beta_ref[0][None, :]


def kernel(word_embeddings, token_type_ids, type_embeddings, position_embeddings,
           gamma, beta):
    b, s, d = word_embeddings.shape
    v = type_embeddings.shape[0]
    blk = 1024
    nblk = s // blk

    ids3 = token_type_ids.astype(jnp.int32).reshape(b * nblk, 1, blk)
    pos = position_embeddings[:s]
    gamma2 = gamma.reshape(1, d)
    beta2 = beta.reshape(1, d)

    out = pl.pallas_call(
        _fused_body,
        grid=(nblk, b),
        in_specs=[
            pl.BlockSpec((1, 1, blk), lambda j, i, n=nblk: (i * n + j, 0, 0)),
            pl.BlockSpec((1, blk, d), lambda j, i: (i, j, 0)),
            pl.BlockSpec((blk, d), lambda j, i: (j, 0)),
            pl.BlockSpec((v, d), lambda j, i: (0, 0)),
            pl.BlockSpec((1, d), lambda j, i: (0, 0)),
            pl.BlockSpec((1, d), lambda j, i: (0, 0)),
        ],
        out_specs=pl.BlockSpec((1, blk, d), lambda j, i: (i, j, 0)),
        out_shape=jax.ShapeDtypeStruct((b, s, d), jnp.float32),
    )(ids3, word_embeddings, pos, type_embeddings, gamma2, beta2)
    return out


# speedup vs baseline: 3.5268x; 1.0873x over previous
import jax
import jax.numpy as jnp
from jax.experimental import pallas as pl

_EPS = 1e-12


def _fused_body(ids_ref, word_ref, pos_ref, type_ref, gamma_ref, beta_ref, out_ref):
    ids = ids_ref[0, 0, :]
    t = ids.shape[0]
    v = type_ref.shape[0]
    onehot = (ids[:, None] == jax.lax.broadcasted_iota(jnp.int32, (t, v), 1)
              ).astype(jnp.float32)
    typ = jnp.dot(onehot, type_ref[...], preferred_element_type=jnp.float32)
    x = word_ref[0] + pos_ref[...] + typ
    mean = jnp.mean(x, axis=-1, keepdims=True)
    cent = x - mean
    var = jnp.mean(cent * cent, axis=-1, keepdims=True)
    normed = cent * jax.lax.rsqrt(var + _EPS)
    out_ref[0] = normed * gamma_ref[0][None, :] + beta_ref[0][None, :]


def kernel(word_embeddings, token_type_ids, type_embeddings, position_embeddings,
           gamma, beta):
    b, s, d = word_embeddings.shape
    v = type_embeddings.shape[0]
    blk = 2048
    nblk = s // blk

    ids3 = token_type_ids.astype(jnp.int32).reshape(b * nblk, 1, blk)
    pos = position_embeddings[:s]
    gamma2 = gamma.reshape(1, d)
    beta2 = beta.reshape(1, d)

    out = pl.pallas_call(
        _fused_body,
        grid=(nblk, b),
        in_specs=[
            pl.BlockSpec((1, 1, blk), lambda j, i, n=nblk: (i * n + j, 0, 0)),
            pl.BlockSpec((1, blk, d), lambda j, i: (i, j, 0)),
            pl.BlockSpec((blk, d), lambda j, i: (j, 0)),
            pl.BlockSpec((v, d), lambda j, i: (0, 0)),
            pl.BlockSpec((1, d), lambda j, i: (0, 0)),
            pl.BlockSpec((1, d), lambda j, i: (0, 0)),
        ],
        out_specs=pl.BlockSpec((1, blk, d), lambda j, i: (i, j, 0)),
        out_shape=jax.ShapeDtypeStruct((b, s, d), jnp.float32),
    )(ids3, word_embeddings, pos, type_embeddings, gamma2, beta2)
    return out
